# baseline (speedup 1.0000x reference)
import functools

import jax
import jax.numpy as jnp
from jax import lax
from jax.experimental import pallas as pl
from jax.experimental.pallas import tpu as pltpu
from jax.experimental.pallas import tpu_sc as plsc


def _sc_partials(eflat, feat_table):
    E = eflat.shape[0] // 2
    N, D = feat_table.shape
    CH = 128
    NCH = E // CH
    NC, NS = 2, 16
    PC = NCH // NC
    TPC = -(-PC // NS)
    RB = 80
    NRC = N // RB
    TRC = -(-NRC // NS)
    DT = N // 10

    mesh = plsc.VectorSubcoreMesh(core_axis_name="c", subcore_axis_name="s")

    @functools.partial(
        pl.kernel,
        out_type=(
            jax.ShapeDtypeStruct((NC, N, D), jnp.float32),
            jax.ShapeDtypeStruct((NC * N,), jnp.float32),
        ),
        mesh=mesh,
        scratch_types=(
            pltpu.VMEM((CH,), jnp.int32),
            pltpu.VMEM((1, CH), jnp.int32),
            pltpu.VMEM((CH,), jnp.int32),
            pltpu.VMEM((1, CH), jnp.int32),
            pltpu.VMEM((CH,), jnp.int32),
            pltpu.VMEM((1, CH), jnp.int32),
            pltpu.VMEM((CH, D), jnp.float32),
            pltpu.VMEM((CH, D), jnp.float32),
            pltpu.VMEM((CH, D), jnp.float32),
            pltpu.VMEM((CH,), jnp.float32),
            pltpu.VMEM_SHARED((N, D), jnp.float32),
            pltpu.VMEM_SHARED((N,), jnp.float32),
            pltpu.SemaphoreType.DMA,
            pltpu.SemaphoreType.DMA,
            pltpu.SemaphoreType.DMA,
            pltpu.SemaphoreType.DMA,
            pltpu.SemaphoreType.DMA,
            pltpu.SemaphoreType.DMA,
            pltpu.SemaphoreType.DMA,
            pltpu.SemaphoreType.DMA,
            pltpu.SemaphoreType.DMA,
            pltpu.SemaphoreType.DMA,
            pltpu.SemaphoreType.DMA,
            pltpu.SemaphoreType.DMA,
            pltpu.SemaphoreType.DMA,
            pltpu.SemaphoreType.DMA,
            pltpu.SemaphoreType.DMA,
        ),
    )
    def sc_kernel(e_ref, feat_ref, part_ref, degp_ref,
                  srcA, dstA, srcB, dstB, srcC, dstC,
                  rowsA, rowsB, rowsC, onesv,
                  agg_sh, deg_sh, gA, gB, gC, sA, sB, sC, dA, dB, dC,
                  iAs, iAd, iBs, iBd, iCs, iCd):
        c = lax.axis_index("c")
        s = lax.axis_index("s")

        zero16 = jnp.zeros((16,), jnp.float32)
        one16 = jnp.ones((16,), jnp.float32)
        for j in range(CH // 16):
            onesv[pl.ds(j * 16, 16)] = one16

        def zrow(i, carry):
            for j in range(D // 16):
                rowsA[i, pl.ds(j * 16, 16)] = zero16
            return carry
        lax.fori_loop(0, CH, zrow, 0)

        def zchunk(t, carry):
            idx = s + NS * t

            @pl.when(idx < NRC)
            def _():
                pltpu.sync_copy(rowsA.at[pl.ds(0, RB)],
                                agg_sh.at[pl.ds(idx * RB, RB)])
                pltpu.sync_copy(rowsA.at[0, pl.ds(0, RB)],
                                deg_sh.at[pl.ds(idx * RB, RB)])
            return carry
        lax.fori_loop(0, TRC, zchunk, 0)

        plsc.subcore_barrier()

        base = c * PC

        def pf_src(k, sv, sem):
            pltpu.async_copy(e_ref.at[pl.ds(E + (base + k) * CH, CH)], sv, sem)

        def wait_src(k, sv, sem):
            pltpu.make_async_copy(e_ref.at[pl.ds(E + (base + k) * CH, CH)],
                                  sv, sem).wait()

        def pf_dst(k, dv, sem):
            pltpu.async_copy(e_ref.at[pl.ds((base + k) * CH, CH)],
                             dv.at[0], sem)

        def wait_dst(k, dv, sem):
            pltpu.make_async_copy(e_ref.at[pl.ds((base + k) * CH, CH)],
                                  dv.at[0], sem).wait()

        def start_gather(sv, rv, sem):
            pltpu.async_copy(feat_ref.at[sv], rv, sem)

        def wait_gather(sv, rv, sem):
            pltpu.make_async_copy(feat_ref.at[sv], rv, sem).wait()

        def start_scatter(dv, rv, sem_r, sem_c):
            pltpu.async_copy(rv, agg_sh.at[dv.at[0]], sem_r, add=True)
            pltpu.async_copy(onesv, deg_sh.at[dv.at[0]], sem_c, add=True)

        def wait_scatter(dv, rv, sem_r, sem_c):
            pltpu.make_async_copy(rv, agg_sh.at[dv.at[0]], sem_r).wait()
            pltpu.make_async_copy(onesv, deg_sh.at[dv.at[0]], sem_c).wait()

        BUFS = ((srcA, dstA, rowsA, gA, sA, dA, iAs, iAd),
                (srcB, dstB, rowsB, gB, sB, dB, iBs, iBd),
                (srcC, dstC, rowsC, gC, sC, dC, iCs, iCd))

        def k_of(m):
            return s + NS * m

        for i in range(3):
            sv, dv, rv, g, _, _, isem, idsem = BUFS[i]
            pf_src(k_of(i), sv, isem)
            wait_src(k_of(i), sv, isem)
            pf_dst(k_of(i), dv, idsem)
            start_gather(sv, rv, g)

        def tri_body(u, carry):
            m0 = 3 * u

            def scat_step(i):
                sv, dv, rv, g, sr, sd, isem, idsem = BUFS[i]
                m = m0 + i
                k = k_of(m)

                @pl.when(k < PC)
                def _():
                    wait_gather(sv, rv, g)

                    @pl.when(k_of(m + 3) < PC)
                    def _():
                        pf_src(k_of(m + 3), sv, isem)

                    wait_dst(k, dv, idsem)
                    start_scatter(dv, rv, sr, sd)

            def drain_step(i):
                sv, dv, rv, g, sr, sd, isem, idsem = BUFS[i]
                m = m0 + i
                k = k_of(m)

                @pl.when(k < PC)
                def _():
                    wait_scatter(dv, rv, sr, sd)

                    @pl.when(k_of(m + 3) < PC)
                    def _():
                        pf_dst(k_of(m + 3), dv, idsem)
                        wait_src(k_of(m + 3), sv, isem)
                        start_gather(sv, rv, g)

            scat_step(0)
            scat_step(1)
            drain_step(0)
            scat_step(2)
            drain_step(1)
            drain_step(2)
            return carry
        lax.fori_loop(0, (TPC + 2) // 3, tri_body, 0)

        plsc.subcore_barrier()

        def wchunk(t, carry):
            idx = s + NS * t

            @pl.when(idx < NRC)
            def _():
                r0 = idx * RB
                pltpu.sync_copy(agg_sh.at[pl.ds(r0, RB)],
                                rowsA.at[pl.ds(0, RB)])
                pltpu.sync_copy(rowsA.at[pl.ds(0, RB)],
                                part_ref.at[c, pl.ds(r0, RB)])
                pltpu.sync_copy(deg_sh.at[pl.ds(r0, RB)],
                                rowsB.at[0, pl.ds(0, RB)])
                pltpu.sync_copy(rowsB.at[0, pl.ds(0, RB)],
                                degp_ref.at[pl.ds(c * N + r0, RB)])
            return carry
        lax.fori_loop(0, TRC, wchunk, 0)

    return sc_kernel(eflat, feat_table)


def _combine(part, degp, feat_table):
    N, D = feat_table.shape
    R = 1000

    def body(p_ref, d_ref, f_ref, o_ref):
        num = p_ref[0] + p_ref[1] + f_ref[...]
        deg = d_ref[0] + d_ref[1] + 1.0
        o_ref[...] = num / deg

    return pl.pallas_call(
        body,
        grid=(N // R,),
        in_specs=[
            pl.BlockSpec((2, R, D), lambda i: (0, i, 0)),
            pl.BlockSpec((2, R, 1), lambda i: (0, i, 0)),
            pl.BlockSpec((R, D), lambda i: (i, 0)),
        ],
        out_specs=pl.BlockSpec((R, D), lambda i: (i, 0)),
        out_shape=jax.ShapeDtypeStruct((N, D), jnp.float32),
    )(part, degp.reshape(2, N, 1), feat_table)


def kernel(nodes, edge_index, feat_table):
    part, degp = _sc_partials(edge_index.reshape(-1), feat_table)
    return _combine(part, degp, feat_table)


# speedup vs baseline: 1.0052x; 1.0052x over previous
import functools

import jax
import jax.numpy as jnp
from jax import lax
from jax.experimental import pallas as pl
from jax.experimental.pallas import tpu as pltpu
from jax.experimental.pallas import tpu_sc as plsc


def _sc_partials(eflat, feat_table):
    E = eflat.shape[0] // 2
    N, D = feat_table.shape
    CH = 128
    NCH = E // CH
    NC, NS = 2, 16
    PC = NCH // NC
    TPC = -(-PC // NS)
    RB = 80
    NRC = N // RB
    TRC = -(-NRC // NS)
    DT = N // 10

    mesh = plsc.VectorSubcoreMesh(core_axis_name="c", subcore_axis_name="s")

    @functools.partial(
        pl.kernel,
        out_type=(
            jax.ShapeDtypeStruct((NC, N, D), jnp.float32),
            jax.ShapeDtypeStruct((NC * N,), jnp.float32),
        ),
        mesh=mesh,
        scratch_types=(
            pltpu.VMEM((CH,), jnp.int32),
            pltpu.VMEM((1, CH), jnp.int32),
            pltpu.VMEM((CH,), jnp.int32),
            pltpu.VMEM((1, CH), jnp.int32),
            pltpu.VMEM((CH, D), jnp.float32),
            pltpu.VMEM((CH, D), jnp.float32),
            pltpu.VMEM((CH,), jnp.float32),
            pltpu.VMEM((1024,), jnp.float32),
            pltpu.VMEM_SHARED((N, D), jnp.float32),
            pltpu.VMEM_SHARED((N,), jnp.float32),
            pltpu.SemaphoreType.DMA,
            pltpu.SemaphoreType.DMA,
            pltpu.SemaphoreType.DMA,
            pltpu.SemaphoreType.DMA,
            pltpu.SemaphoreType.DMA,
            pltpu.SemaphoreType.DMA,
            pltpu.SemaphoreType.DMA,
            pltpu.SemaphoreType.DMA,
            pltpu.SemaphoreType.DMA,
            pltpu.SemaphoreType.DMA,
        ),
    )
    def sc_kernel(e_ref, feat_ref, part_ref, degp_ref,
                  srcA, dstA, srcB, dstB, rowsA, rowsB, onesv, zv,
                  agg_sh, deg_sh, gA, gB, sA, sB, dA, dB,
                  iAs, iAd, iBs, iBd):
        c = lax.axis_index("c")
        s = lax.axis_index("s")

        zero16 = jnp.zeros((16,), jnp.float32)
        one16 = jnp.ones((16,), jnp.float32)
        for j in range(CH // 16):
            onesv[pl.ds(j * 16, 16)] = one16
        for j in range(1024 // 16):
            zv[pl.ds(j * 16, 16)] = zero16

        def zrow(i, carry):
            for j in range(D // 16):
                rowsA[i, pl.ds(j * 16, 16)] = zero16
            return carry
        lax.fori_loop(0, CH, zrow, 0)

        def zchunk(t, carry):
            idx = s + NS * t

            @pl.when(idx < NRC)
            def _():
                pltpu.sync_copy(rowsA.at[pl.ds(0, RB)],
                                agg_sh.at[pl.ds(idx * RB, RB)])
            return carry
        lax.fori_loop(0, TRC, zchunk, 0)

        @pl.when(s < 10)
        def _():
            pltpu.sync_copy(zv.at[pl.ds(0, DT)], deg_sh.at[pl.ds(s * DT, DT)])

        plsc.subcore_barrier()

        base = c * PC

        def pf_src(k, sv, sem):
            pltpu.async_copy(e_ref.at[pl.ds(E + (base + k) * CH, CH)], sv, sem)

        def wait_src(k, sv, sem):
            pltpu.make_async_copy(e_ref.at[pl.ds(E + (base + k) * CH, CH)],
                                  sv, sem).wait()

        def pf_dst(k, dv, sem):
            pltpu.async_copy(e_ref.at[pl.ds((base + k) * CH, CH)],
                             dv.at[0], sem)

        def wait_dst(k, dv, sem):
            pltpu.make_async_copy(e_ref.at[pl.ds((base + k) * CH, CH)],
                                  dv.at[0], sem).wait()

        def start_gather(sv, rv, sem):
            pltpu.async_copy(feat_ref.at[sv], rv, sem)

        def wait_gather(sv, rv, sem):
            pltpu.make_async_copy(feat_ref.at[sv], rv, sem).wait()

        def start_scatter(dv, rv, sem_r, sem_c):
            pltpu.async_copy(rv, agg_sh.at[dv.at[0]], sem_r, add=True)
            pltpu.async_copy(onesv, deg_sh.at[dv.at[0]], sem_c, add=True)

        def wait_scatter(dv, rv, sem_r, sem_c):
            pltpu.make_async_copy(rv, agg_sh.at[dv.at[0]], sem_r).wait()
            pltpu.make_async_copy(onesv, deg_sh.at[dv.at[0]], sem_c).wait()

        k0p = s
        k1p = s + NS
        pf_src(k0p, srcA, iAs)
        wait_src(k0p, srcA, iAs)
        pf_dst(k0p, dstA, iAd)
        start_gather(srcA, rowsA, gA)
        pf_src(k1p, srcB, iBs)
        wait_src(k1p, srcB, iBs)
        pf_dst(k1p, dstB, iBd)
        start_gather(srcB, rowsB, gB)

        def pair_body(u, carry):
            k0 = s + NS * (2 * u)
            k1 = s + NS * (2 * u + 1)
            k2 = s + NS * (2 * u + 2)
            k3 = s + NS * (2 * u + 3)

            @pl.when(k0 < PC)
            def _():
                wait_gather(srcA, rowsA, gA)

                @pl.when(k2 < PC)
                def _():
                    pf_src(k2, srcA, iAs)

                wait_dst(k0, dstA, iAd)
                start_scatter(dstA, rowsA, sA, dA)

            @pl.when(k1 < PC)
            def _():
                wait_gather(srcB, rowsB, gB)

                @pl.when(k3 < PC)
                def _():
                    pf_src(k3, srcB, iBs)

            @pl.when(k0 < PC)
            def _():
                wait_scatter(dstA, rowsA, sA, dA)

                @pl.when(k2 < PC)
                def _():
                    pf_dst(k2, dstA, iAd)
                    wait_src(k2, srcA, iAs)
                    start_gather(srcA, rowsA, gA)

            @pl.when(k1 < PC)
            def _():
                wait_dst(k1, dstB, iBd)
                start_scatter(dstB, rowsB, sB, dB)
                wait_scatter(dstB, rowsB, sB, dB)

                @pl.when(k3 < PC)
                def _():
                    pf_dst(k3, dstB, iBd)
                    wait_src(k3, srcB, iBs)
                    start_gather(srcB, rowsB, gB)
            return carry
        lax.fori_loop(0, (TPC + 1) // 2, pair_body, 0)

        plsc.subcore_barrier()

        def wchunk(t, carry):
            idx = s + NS * t

            @pl.when(idx < NRC)
            def _():
                r0 = idx * RB
                pltpu.sync_copy(agg_sh.at[pl.ds(r0, RB)],
                                rowsA.at[pl.ds(0, RB)])
                pltpu.sync_copy(rowsA.at[pl.ds(0, RB)],
                                part_ref.at[c, pl.ds(r0, RB)])
            return carry
        lax.fori_loop(0, TRC, wchunk, 0)

        @pl.when(s < 10)
        def _():
            pltpu.sync_copy(deg_sh.at[pl.ds(s * DT, DT)], zv.at[pl.ds(0, DT)])
            pltpu.sync_copy(zv.at[pl.ds(0, DT)],
                            degp_ref.at[pl.ds(c * N + s * DT, DT)])

    return sc_kernel(eflat, feat_table)


def _combine(part, degp, feat_table):
    N, D = feat_table.shape
    R = 1000

    def body(p_ref, d_ref, f_ref, o_ref):
        num = p_ref[0] + p_ref[1] + f_ref[...]
        deg = d_ref[0] + d_ref[1] + 1.0
        o_ref[...] = num / deg

    return pl.pallas_call(
        body,
        grid=(N // R,),
        in_specs=[
            pl.BlockSpec((2, R, D), lambda i: (0, i, 0)),
            pl.BlockSpec((2, R, 1), lambda i: (0, i, 0)),
            pl.BlockSpec((R, D), lambda i: (i, 0)),
        ],
        out_specs=pl.BlockSpec((R, D), lambda i: (i, 0)),
        out_shape=jax.ShapeDtypeStruct((N, D), jnp.float32),
    )(part, degp.reshape(2, N, 1), feat_table)


def kernel(nodes, edge_index, feat_table):
    part, degp = _sc_partials(edge_index.reshape(-1), feat_table)
    return _combine(part, degp, feat_table)


# speedup vs baseline: 1.0115x; 1.0063x over previous
import functools

import jax
import jax.numpy as jnp
from jax import lax
from jax.experimental import pallas as pl
from jax.experimental.pallas import tpu as pltpu
from jax.experimental.pallas import tpu_sc as plsc


def _sc_partials(eflat, feat_table):
    E = eflat.shape[0] // 2
    N, D = feat_table.shape
    CH = 128
    NCH = E // CH
    NC, NS = 2, 16
    PC = NCH // NC
    TPC = -(-PC // NS)
    RB = 80
    NRC = N // RB
    TRC = -(-NRC // NS)
    DT = N // 10

    mesh = plsc.VectorSubcoreMesh(core_axis_name="c", subcore_axis_name="s")

    @functools.partial(
        pl.kernel,
        out_type=(
            jax.ShapeDtypeStruct((NC, N, D), jnp.float32),
            jax.ShapeDtypeStruct((NC * N,), jnp.float32),
        ),
        mesh=mesh,
        scratch_types=(
            pltpu.VMEM((CH,), jnp.int32),
            pltpu.VMEM((1, CH), jnp.int32),
            pltpu.VMEM((CH,), jnp.int32),
            pltpu.VMEM((1, CH), jnp.int32),
            pltpu.VMEM((CH, D), jnp.float32),
            pltpu.VMEM((CH, D), jnp.float32),
            pltpu.VMEM((CH,), jnp.float32),
            pltpu.VMEM((1024,), jnp.float32),
            pltpu.VMEM_SHARED((N, D), jnp.float32),
            pltpu.VMEM_SHARED((N,), jnp.float32),
            pltpu.SemaphoreType.DMA,
            pltpu.SemaphoreType.DMA,
            pltpu.SemaphoreType.DMA,
            pltpu.SemaphoreType.DMA,
            pltpu.SemaphoreType.DMA,
            pltpu.SemaphoreType.DMA,
            pltpu.SemaphoreType.DMA,
            pltpu.SemaphoreType.DMA,
            pltpu.SemaphoreType.DMA,
            pltpu.SemaphoreType.DMA,
        ),
    )
    def sc_kernel(e_ref, feat_ref, part_ref, degp_ref,
                  srcA, dstA, srcB, dstB, rowsA, rowsB, onesv, zv,
                  agg_sh, deg_sh, gA, gB, sA, sB, dA, dB,
                  iAs, iAd, iBs, iBd):
        c = lax.axis_index("c")
        s = lax.axis_index("s")

        zero16 = jnp.zeros((16,), jnp.float32)
        one16 = jnp.ones((16,), jnp.float32)
        for j in range(CH // 16):
            onesv[pl.ds(j * 16, 16)] = one16
        for j in range(1024 // 16):
            zv[pl.ds(j * 16, 16)] = zero16

        def zrow(i, carry):
            for j in range(D // 16):
                rowsA[i, pl.ds(j * 16, 16)] = zero16
            return carry
        lax.fori_loop(0, CH, zrow, 0)

        def zchunk(t, carry):
            idx = s + NS * t

            @pl.when(idx < NRC)
            def _():
                pltpu.sync_copy(rowsA.at[pl.ds(0, RB)],
                                agg_sh.at[pl.ds(idx * RB, RB)])
            return carry
        lax.fori_loop(0, TRC, zchunk, 0)

        @pl.when(s < 10)
        def _():
            pltpu.sync_copy(zv.at[pl.ds(0, DT)], deg_sh.at[pl.ds(s * DT, DT)])

        plsc.subcore_barrier()

        base = c * PC

        def pf_src(k, sv, sem):
            pltpu.async_copy(e_ref.at[pl.ds(E + (base + k) * CH, CH)], sv, sem)

        def wait_src(k, sv, sem):
            pltpu.make_async_copy(e_ref.at[pl.ds(E + (base + k) * CH, CH)],
                                  sv, sem).wait()

        def pf_dst(k, dv, sem):
            pltpu.async_copy(e_ref.at[pl.ds((base + k) * CH, CH)],
                             dv.at[0], sem)

        def wait_dst(k, dv, sem):
            pltpu.make_async_copy(e_ref.at[pl.ds((base + k) * CH, CH)],
                                  dv.at[0], sem).wait()

        def start_gather(sv, rv, sem):
            pltpu.async_copy(feat_ref.at[sv], rv, sem)

        def wait_gather(sv, rv, sem):
            pltpu.make_async_copy(feat_ref.at[sv], rv, sem).wait()

        def start_scatter(dv, rv, sem_r, sem_c):
            pltpu.async_copy(rv, agg_sh.at[dv.at[0]], sem_r, add=True)
            pltpu.async_copy(onesv, deg_sh.at[dv.at[0]], sem_c, add=True)

        def wait_scatter(dv, rv, sem_r, sem_c):
            pltpu.make_async_copy(rv, agg_sh.at[dv.at[0]], sem_r).wait()
            pltpu.make_async_copy(onesv, deg_sh.at[dv.at[0]], sem_c).wait()

        k0p = s
        k1p = s + NS
        pf_src(k0p, srcA, iAs)
        wait_src(k0p, srcA, iAs)
        pf_dst(k0p, dstA, iAd)
        start_gather(srcA, rowsA, gA)
        pf_src(k1p, srcB, iBs)
        wait_src(k1p, srcB, iBs)
        pf_dst(k1p, dstB, iBd)
        start_gather(srcB, rowsB, gB)

        def pair_body(u, carry):
            k0 = s + NS * (2 * u)
            k1 = s + NS * (2 * u + 1)
            k2 = s + NS * (2 * u + 2)
            k3 = s + NS * (2 * u + 3)

            @pl.when(k0 < PC)
            def _():
                wait_gather(srcA, rowsA, gA)

                @pl.when(k2 < PC)
                def _():
                    pf_src(k2, srcA, iAs)

                wait_dst(k0, dstA, iAd)
                start_scatter(dstA, rowsA, sA, dA)

            @pl.when(k1 < PC)
            def _():
                wait_gather(srcB, rowsB, gB)

                @pl.when(k3 < PC)
                def _():
                    pf_src(k3, srcB, iBs)

            @pl.when(k0 < PC)
            def _():
                wait_scatter(dstA, rowsA, sA, dA)

                @pl.when(k2 < PC)
                def _():
                    pf_dst(k2, dstA, iAd)
                    wait_src(k2, srcA, iAs)
                    start_gather(srcA, rowsA, gA)

            @pl.when(k1 < PC)
            def _():
                wait_dst(k1, dstB, iBd)
                start_scatter(dstB, rowsB, sB, dB)
                wait_scatter(dstB, rowsB, sB, dB)

                @pl.when(k3 < PC)
                def _():
                    pf_dst(k3, dstB, iBd)
                    wait_src(k3, srcB, iBs)
                    start_gather(srcB, rowsB, gB)
            return carry
        lax.fori_loop(0, (TPC + 1) // 2, pair_body, 0)

        plsc.subcore_barrier()

        def wchunk(t, carry):
            idx = s + NS * t

            @pl.when(idx < NRC)
            def _():
                r0 = idx * RB
                pltpu.sync_copy(agg_sh.at[pl.ds(r0, RB)],
                                rowsA.at[pl.ds(0, RB)])
                pltpu.sync_copy(rowsA.at[pl.ds(0, RB)],
                                part_ref.at[c, pl.ds(r0, RB)])
            return carry
        lax.fori_loop(0, TRC, wchunk, 0)

        @pl.when(s < 10)
        def _():
            pltpu.sync_copy(deg_sh.at[pl.ds(s * DT, DT)], zv.at[pl.ds(0, DT)])
            pltpu.sync_copy(zv.at[pl.ds(0, DT)],
                            degp_ref.at[pl.ds(c * N + s * DT, DT)])

    return sc_kernel(eflat, feat_table)


def _combine(part, degp, feat_table):
    N, D = feat_table.shape
    R = 2000

    def body(p_ref, d_ref, f_ref, o_ref):
        num = p_ref[0] + p_ref[1] + f_ref[...]
        deg = d_ref[0] + d_ref[1] + 1.0
        o_ref[...] = num / deg

    return pl.pallas_call(
        body,
        grid=(N // R,),
        in_specs=[
            pl.BlockSpec((2, R, D), lambda i: (0, i, 0)),
            pl.BlockSpec((2, R, 1), lambda i: (0, i, 0)),
            pl.BlockSpec((R, D), lambda i: (i, 0)),
        ],
        out_specs=pl.BlockSpec((R, D), lambda i: (i, 0)),
        out_shape=jax.ShapeDtypeStruct((N, D), jnp.float32),
    )(part, degp.reshape(2, N, 1), feat_table)


def kernel(nodes, edge_index, feat_table):
    part, degp = _sc_partials(edge_index.reshape(-1), feat_table)
    return _combine(part, degp, feat_table)


# speedup vs baseline: 1.0226x; 1.0110x over previous
import functools

import jax
import jax.numpy as jnp
from jax import lax
from jax.experimental import pallas as pl
from jax.experimental.pallas import tpu as pltpu
from jax.experimental.pallas import tpu_sc as plsc


def _sc_partials(eflat, feat_table):
    E = eflat.shape[0] // 2
    N, D = feat_table.shape
    CH = 128
    NCH = E // CH
    NC, NS = 2, 16
    PC = NCH // NC
    TPC = -(-PC // NS)
    RB = 80
    NRC = N // RB
    TRC = -(-NRC // NS)
    DT = N // 10

    mesh = plsc.VectorSubcoreMesh(core_axis_name="c", subcore_axis_name="s")

    @functools.partial(
        pl.kernel,
        out_type=(
            jax.ShapeDtypeStruct((NC, N, D), jnp.float32),
            jax.ShapeDtypeStruct((NC * N,), jnp.float32),
        ),
        mesh=mesh,
        scratch_types=(
            pltpu.VMEM((CH,), jnp.int32),
            pltpu.VMEM((1, CH), jnp.int32),
            pltpu.VMEM((CH,), jnp.int32),
            pltpu.VMEM((1, CH), jnp.int32),
            pltpu.VMEM((CH, D), jnp.float32),
            pltpu.VMEM((CH, D), jnp.float32),
            pltpu.VMEM((CH,), jnp.float32),
            pltpu.VMEM((1024,), jnp.float32),
            pltpu.VMEM((80, 128), jnp.float32),
            pltpu.VMEM_SHARED((N, D), jnp.float32),
            pltpu.VMEM_SHARED((N,), jnp.float32),
            pltpu.SemaphoreType.DMA,
            pltpu.SemaphoreType.DMA,
            pltpu.SemaphoreType.DMA,
            pltpu.SemaphoreType.DMA,
            pltpu.SemaphoreType.DMA,
            pltpu.SemaphoreType.DMA,
            pltpu.SemaphoreType.DMA,
            pltpu.SemaphoreType.DMA,
            pltpu.SemaphoreType.DMA,
            pltpu.SemaphoreType.DMA,
        ),
    )
    def sc_kernel(e_ref, feat_ref, part_ref, degp_ref,
                  srcA, dstA, srcB, dstB, rowsA, rowsB, onesv, zv, zbuf,
                  agg_sh, deg_sh, gA, gB, sA, sB, dA, dB,
                  iAs, iAd, iBs, iBd):
        c = lax.axis_index("c")
        s = lax.axis_index("s")

        zero16 = jnp.zeros((16,), jnp.float32)
        one16 = jnp.ones((16,), jnp.float32)
        for j in range(CH // 16):
            onesv[pl.ds(j * 16, 16)] = one16
        for j in range(1024 // 16):
            zv[pl.ds(j * 16, 16)] = zero16

        def zrow(i, carry):
            for j in range(D // 16):
                zbuf[i, pl.ds(j * 16, 16)] = zero16
            return carry
        lax.fori_loop(0, RB, zrow, 0)

        base = c * PC

        def pf_src(k, sv, sem):
            pltpu.async_copy(e_ref.at[pl.ds(E + (base + k) * CH, CH)], sv, sem)

        def wait_src(k, sv, sem):
            pltpu.make_async_copy(e_ref.at[pl.ds(E + (base + k) * CH, CH)],
                                  sv, sem).wait()

        def pf_dst(k, dv, sem):
            pltpu.async_copy(e_ref.at[pl.ds((base + k) * CH, CH)],
                             dv.at[0], sem)

        def wait_dst(k, dv, sem):
            pltpu.make_async_copy(e_ref.at[pl.ds((base + k) * CH, CH)],
                                  dv.at[0], sem).wait()

        def start_gather(sv, rv, sem):
            pltpu.async_copy(feat_ref.at[sv], rv, sem)

        def wait_gather(sv, rv, sem):
            pltpu.make_async_copy(feat_ref.at[sv], rv, sem).wait()

        def start_scatter(dv, rv, sem_r, sem_c):
            pltpu.async_copy(rv, agg_sh.at[dv.at[0]], sem_r, add=True)
            pltpu.async_copy(onesv, deg_sh.at[dv.at[0]], sem_c, add=True)

        def wait_scatter(dv, rv, sem_r, sem_c):
            pltpu.make_async_copy(rv, agg_sh.at[dv.at[0]], sem_r).wait()
            pltpu.make_async_copy(onesv, deg_sh.at[dv.at[0]], sem_c).wait()

        k0p = s
        k1p = s + NS
        pf_src(k0p, srcA, iAs)
        wait_src(k0p, srcA, iAs)
        pf_dst(k0p, dstA, iAd)
        start_gather(srcA, rowsA, gA)
        pf_src(k1p, srcB, iBs)
        wait_src(k1p, srcB, iBs)
        pf_dst(k1p, dstB, iBd)
        start_gather(srcB, rowsB, gB)

        def zchunk(t, carry):
            idx = s + NS * t

            @pl.when(idx < NRC)
            def _():
                pltpu.sync_copy(zbuf.at[pl.ds(0, RB)],
                                agg_sh.at[pl.ds(idx * RB, RB)])
            return carry
        lax.fori_loop(0, TRC, zchunk, 0)

        @pl.when(s < 10)
        def _():
            pltpu.sync_copy(zv.at[pl.ds(0, DT)], deg_sh.at[pl.ds(s * DT, DT)])

        plsc.subcore_barrier()

        def pair_body(u, carry):
            k0 = s + NS * (2 * u)
            k1 = s + NS * (2 * u + 1)
            k2 = s + NS * (2 * u + 2)
            k3 = s + NS * (2 * u + 3)

            @pl.when(k0 < PC)
            def _():
                wait_gather(srcA, rowsA, gA)

                @pl.when(k2 < PC)
                def _():
                    pf_src(k2, srcA, iAs)

                wait_dst(k0, dstA, iAd)
                start_scatter(dstA, rowsA, sA, dA)

            @pl.when(k1 < PC)
            def _():
                wait_gather(srcB, rowsB, gB)

                @pl.when(k3 < PC)
                def _():
                    pf_src(k3, srcB, iBs)

            @pl.when(k0 < PC)
            def _():
                wait_scatter(dstA, rowsA, sA, dA)

                @pl.when(k2 < PC)
                def _():
                    pf_dst(k2, dstA, iAd)
                    wait_src(k2, srcA, iAs)
                    start_gather(srcA, rowsA, gA)

            @pl.when(k1 < PC)
            def _():
                wait_dst(k1, dstB, iBd)
                start_scatter(dstB, rowsB, sB, dB)
                wait_scatter(dstB, rowsB, sB, dB)

                @pl.when(k3 < PC)
                def _():
                    pf_dst(k3, dstB, iBd)
                    wait_src(k3, srcB, iBs)
                    start_gather(srcB, rowsB, gB)
            return carry
        lax.fori_loop(0, (TPC + 1) // 2, pair_body, 0)

        plsc.subcore_barrier()

        def wchunk(t, carry):
            idx = s + NS * t

            @pl.when(idx < NRC)
            def _():
                r0 = idx * RB
                pltpu.sync_copy(agg_sh.at[pl.ds(r0, RB)],
                                rowsA.at[pl.ds(0, RB)])
                pltpu.sync_copy(rowsA.at[pl.ds(0, RB)],
                                part_ref.at[c, pl.ds(r0, RB)])
            return carry
        lax.fori_loop(0, TRC, wchunk, 0)

        @pl.when(s < 10)
        def _():
            pltpu.sync_copy(deg_sh.at[pl.ds(s * DT, DT)], zv.at[pl.ds(0, DT)])
            pltpu.sync_copy(zv.at[pl.ds(0, DT)],
                            degp_ref.at[pl.ds(c * N + s * DT, DT)])

    return sc_kernel(eflat, feat_table)


def _combine(part, degp, feat_table):
    N, D = feat_table.shape
    R = 2000

    def body(p_ref, d_ref, f_ref, o_ref):
        num = p_ref[0] + p_ref[1] + f_ref[...]
        deg = d_ref[0] + d_ref[1] + 1.0
        o_ref[...] = num / deg

    return pl.pallas_call(
        body,
        grid=(N // R,),
        in_specs=[
            pl.BlockSpec((2, R, D), lambda i: (0, i, 0)),
            pl.BlockSpec((2, R, 1), lambda i: (0, i, 0)),
            pl.BlockSpec((R, D), lambda i: (i, 0)),
        ],
        out_specs=pl.BlockSpec((R, D), lambda i: (i, 0)),
        out_shape=jax.ShapeDtypeStruct((N, D), jnp.float32),
    )(part, degp.reshape(2, N, 1), feat_table)


def kernel(nodes, edge_index, feat_table):
    part, degp = _sc_partials(edge_index.reshape(-1), feat_table)
    return _combine(part, degp, feat_table)
